# baseline (speedup 1.0000x reference)
import jax
import jax.numpy as jnp
from jax.experimental import pallas as pl
from jax.experimental.pallas import tpu as pltpu

_DIM = 128
_B = 4096


def _enqueue_body(ptr_ref, keys_ref, queue_hbm, out_hbm, knt_vmem, sem):
    ptr = ptr_ref[0]
    slot_blk = ptr // _B

    k = keys_ref[...]
    norm = jnp.sqrt(jnp.sum(k * k, axis=1, keepdims=True))
    knt_vmem[...] = (k / jnp.maximum(norm, 1e-12)).T

    nchunk = queue_hbm.shape[1] // _B
    copies = []
    for j in range(nchunk):
        cols = pl.ds(j * _B, _B)
        hbm_copy = pltpu.make_async_copy(
            queue_hbm.at[:, cols], out_hbm.at[:, cols], sem
        )
        key_copy = pltpu.make_async_copy(knt_vmem, out_hbm.at[:, cols], sem)

        @pl.when(j != slot_blk)
        def _():
            hbm_copy.start()

        @pl.when(j == slot_blk)
        def _():
            key_copy.start()

        copies.append(hbm_copy)
    for c in copies:
        c.wait()


def kernel(keys, queue, ptr, filled):
    keys = keys.astype(jnp.float32)
    b, dim = keys.shape
    dim2, kq = queue.shape
    assert dim == _DIM and dim2 == _DIM and b == _B and kq % _B == 0

    ptr_arr = jnp.asarray(ptr, jnp.int32).reshape(1)

    grid_spec = pltpu.PrefetchScalarGridSpec(
        num_scalar_prefetch=1,
        grid=(1,),
        in_specs=[
            pl.BlockSpec((b, dim), lambda j, p: (0, 0)),
            pl.BlockSpec(memory_space=pl.ANY),
        ],
        out_specs=pl.BlockSpec(memory_space=pl.ANY),
        scratch_shapes=[
            pltpu.VMEM((dim, b), jnp.float32),
            pltpu.SemaphoreType.DMA,
        ],
    )

    new_queue = pl.pallas_call(
        _enqueue_body,
        grid_spec=grid_spec,
        out_shape=jax.ShapeDtypeStruct((dim, kq), jnp.float32),
    )(ptr_arr, keys, queue)

    new_ptr = jnp.reshape((ptr + b) % kq, (1,)).astype(jnp.int32)
    new_filled = jnp.reshape(jnp.minimum(filled + b, kq), (1,)).astype(jnp.int32)
    return new_queue, new_ptr, new_filled


# speedup vs baseline: 41.5230x; 41.5230x over previous
import jax
import jax.numpy as jnp
from jax.experimental import pallas as pl
from jax.experimental.pallas import tpu as pltpu

_DIM = 128
_B = 4096
_ROWS = 16


def _enqueue_body(ptr_ref, keys_ref, queue_ref, out_ref, knt_vmem):
    r = pl.program_id(0)

    @pl.when(r == 0)
    def _normalize():
        k = keys_ref[...]
        norm = jnp.sqrt(jnp.sum(k * k, axis=1, keepdims=True))
        knt_vmem[...] = (k / jnp.maximum(norm, 1e-12)).T

    out_ref[...] = queue_ref[...]
    ptr = pl.multiple_of(ptr_ref[0], 512)
    out_ref[:, pl.ds(ptr, _B)] = knt_vmem[pl.ds(r * _ROWS, _ROWS), :]


def kernel(keys, queue, ptr, filled):
    keys = keys.astype(jnp.float32)
    b, dim = keys.shape
    dim2, kq = queue.shape
    assert dim == _DIM and dim2 == _DIM and b == _B and dim % _ROWS == 0

    ptr_arr = jnp.asarray(ptr, jnp.int32).reshape(1)

    grid_spec = pltpu.PrefetchScalarGridSpec(
        num_scalar_prefetch=1,
        grid=(dim // _ROWS,),
        in_specs=[
            pl.BlockSpec((b, dim), lambda r, p: (0, 0)),
            pl.BlockSpec((_ROWS, kq), lambda r, p: (r, 0)),
        ],
        out_specs=pl.BlockSpec((_ROWS, kq), lambda r, p: (r, 0)),
        scratch_shapes=[pltpu.VMEM((dim, b), jnp.float32)],
    )

    new_queue = pl.pallas_call(
        _enqueue_body,
        grid_spec=grid_spec,
        out_shape=jax.ShapeDtypeStruct((dim, kq), jnp.float32),
    )(ptr_arr, keys, queue)

    new_ptr = jnp.reshape((ptr + b) % kq, (1,)).astype(jnp.int32)
    new_filled = jnp.reshape(jnp.minimum(filled + b, kq), (1,)).astype(jnp.int32)
    return new_queue, new_ptr, new_filled
